# baseline (speedup 1.0000x reference)
import jax
import jax.numpy as jnp
from jax.experimental import pallas as pl
from jax.experimental.pallas import tpu as pltpu

_R = 8192


def _compute(p_ref, tau, a, c):
    mu = p_ref[0]
    t0 = p_ref[5]
    t1 = t0 + p_ref[6]
    t2 = t1 + p_ref[7]
    f_ble = jnp.where(a <= t0, p_ref[1],
            jnp.where(a <= t1, p_ref[2],
            jnp.where(a <= t2, p_ref[3], p_ref[4])))
    f_con = jnp.where(c == 2.0, p_ref[8],
            jnp.where(c == 3.0, p_ref[9], 0.0))
    return 1.0 - jnp.exp(-mu * (tau * f_ble * f_con))


def _body_a(p_ref, tau_ref, a_ref, c_ref, dummy_ref, out_ref):
    out_ref[...] = _compute(p_ref, tau_ref[0], a_ref[0], c_ref[0])


def _body_b(p_ref, tau_ref, a_ref, c_ref, out_ref):
    out_ref[...] = _compute(p_ref, tau_ref[...], a_ref[...], c_ref[...])


def kernel(events, params):
    n = events.shape[0]
    rows = n // 128
    rows_a = 24576
    rows_b = rows - rows_a
    n_a = rows_a * 128
    ga = rows_a // _R

    ev_b = events[n_a:]
    tau_b = ev_b[:, 0].reshape(rows_b, 128)
    a_b = ev_b[:, 1].reshape(rows_b, 128)
    c_b = ev_b[:, 2].reshape(rows_b, 128)

    spec_b = pl.BlockSpec((_R, 128), lambda i: (i, 0))
    out_b = pl.pallas_call(
        _body_b,
        grid=(rows_b // _R,),
        in_specs=[
            pl.BlockSpec(memory_space=pltpu.SMEM),
            spec_b, spec_b, spec_b,
        ],
        out_specs=pl.BlockSpec((_R, 128), lambda i, ga=ga: (ga + i, 0)),
        out_shape=jax.ShapeDtypeStruct((rows, 128), jnp.float32),
    )(params, tau_b, a_b, c_b)

    ev_a = events.reshape(rows, 128, 3).transpose(2, 0, 1)

    def fspec(f):
        return pl.BlockSpec((1, _R, 128), lambda i, f=f: (f, i, 0))

    out = pl.pallas_call(
        _body_a,
        grid=(ga,),
        in_specs=[
            pl.BlockSpec(memory_space=pltpu.SMEM),
            fspec(0), fspec(1), fspec(2),
            pl.BlockSpec(memory_space=pl.ANY),
        ],
        out_specs=pl.BlockSpec((_R, 128), lambda i: (i, 0)),
        out_shape=jax.ShapeDtypeStruct((rows, 128), jnp.float32),
        input_output_aliases={4: 0},
    )(params, ev_a, ev_a, ev_a, out_b)
    return out.reshape(n)


# speedup vs baseline: 1.2537x; 1.2537x over previous
import jax
import jax.numpy as jnp
from jax.experimental import pallas as pl
from jax.experimental.pallas import tpu as pltpu

_R = 8192


def _body(p_ref, tau_ref, a_ref, c_ref, out_ref):
    mu = p_ref[0]
    t0 = p_ref[5]
    t1 = t0 + p_ref[6]
    t2 = t1 + p_ref[7]

    tau = tau_ref[0]
    a = a_ref[0]
    c = c_ref[0]

    f_ble = jnp.where(a <= t0, p_ref[1],
            jnp.where(a <= t1, p_ref[2],
            jnp.where(a <= t2, p_ref[3], p_ref[4])))
    f_con = jnp.where(c == 2.0, p_ref[8],
            jnp.where(c == 3.0, p_ref[9], 0.0))
    r = tau * f_ble * f_con
    out_ref[...] = 1.0 - jnp.exp(-mu * r)


def kernel(events, params):
    n = events.shape[0]
    rows = n // 128
    evt = events.reshape(rows, 128, 3).transpose(2, 0, 1)
    grid = (rows // _R,)

    def fspec(f):
        return pl.BlockSpec((1, _R, 128), lambda i, f=f: (f, i, 0))

    out = pl.pallas_call(
        _body,
        grid=grid,
        in_specs=[
            pl.BlockSpec(memory_space=pltpu.SMEM),
            fspec(0), fspec(1), fspec(2),
        ],
        out_specs=pl.BlockSpec((_R, 128), lambda i: (i, 0)),
        out_shape=jax.ShapeDtypeStruct((rows, 128), jnp.float32),
    )(params, evt, evt, evt)
    return out.reshape(n)


# speedup vs baseline: 1.2574x; 1.0029x over previous
import jax
import jax.numpy as jnp
from jax.experimental import pallas as pl
from jax.experimental.pallas import tpu as pltpu

_R = 8192


def _body(p_ref, ev_ref, out_ref):
    mu = p_ref[0]
    t0 = p_ref[5]
    t1 = t0 + p_ref[6]
    t2 = t1 + p_ref[7]

    tau = ev_ref[0]
    a = ev_ref[1]
    c = ev_ref[2]

    f_ble = jnp.where(a <= t0, p_ref[1],
            jnp.where(a <= t1, p_ref[2],
            jnp.where(a <= t2, p_ref[3], p_ref[4])))
    f_con = jnp.where(c == 2.0, p_ref[8],
            jnp.where(c == 3.0, p_ref[9], 0.0))
    r = tau * f_ble * f_con
    out_ref[...] = 1.0 - jnp.exp(-mu * r)


def kernel(events, params):
    n = events.shape[0]
    rows = n // 128
    evt = events.reshape(rows, 128, 3).transpose(2, 0, 1)
    grid = (rows // _R,)

    out = pl.pallas_call(
        _body,
        grid=grid,
        in_specs=[
            pl.BlockSpec(memory_space=pltpu.SMEM),
            pl.BlockSpec((3, _R, 128), lambda i: (0, i, 0)),
        ],
        out_specs=pl.BlockSpec((_R, 128), lambda i: (i, 0)),
        out_shape=jax.ShapeDtypeStruct((rows, 128), jnp.float32),
    )(params, evt)
    return out.reshape(n)
